# baseline (speedup 1.0000x reference)
import jax
import jax.numpy as jnp
from jax.experimental import pallas as pl
from jax.experimental.pallas import tpu as pltpu

_NUM_TAGS = 32
_SHIFT = 128
_L = _NUM_TAGS * _SHIFT
_D = 512
_B = 8
_K = 8
_LBLK = 512
_NBLK = _L // _LBLK


def _manual_kernel(tags_ref, emb_hbm, out_hbm, mask_ref, buf, in_sem, out_sem):
    tag_ids = jax.lax.broadcasted_iota(jnp.int32, (1, _L), 1) // _SHIFT
    tags = tags_ref[...]
    acc = jnp.zeros((_B, _L), jnp.bool_)
    for k in range(_K):
        acc = acc | (tags[:, k : k + 1] == tag_ids)
    mask_ref[...] = acc.astype(jnp.int32)

    def cp_in(l, s):
        return pltpu.make_async_copy(
            emb_hbm.at[pl.ds(l * _LBLK, _LBLK), :], buf.at[s], in_sem.at[s]
        )

    def cp_out(l, b, s):
        return pltpu.make_async_copy(
            buf.at[s], out_hbm.at[b, pl.ds(l * _LBLK, _LBLK), :], out_sem.at[s]
        )

    cp_in(0, 0).start()
    for l in range(_NBLK):
        s = l % 2
        cp_in(l, s).wait()
        for b in range(_B):
            cp_out(l, b, s).start()
        if l + 1 < _NBLK:
            ns = (l + 1) % 2
            if l >= 1:
                for b in range(_B):
                    cp_out(l - 1, b, ns).wait()
            cp_in(l + 1, ns).start()
    for b in range(_B):
        cp_out(_NBLK - 2, b, (_NBLK - 2) % 2).wait()
    for b in range(_B):
        cp_out(_NBLK - 1, b, (_NBLK - 1) % 2).wait()


def kernel(tags, embedding):
    embed, mask = pl.pallas_call(
        _manual_kernel,
        in_specs=[
            pl.BlockSpec((_B, _K), lambda: (0, 0)),
            pl.BlockSpec(memory_space=pl.ANY),
        ],
        out_specs=[
            pl.BlockSpec(memory_space=pl.ANY),
            pl.BlockSpec((_B, _L), lambda: (0, 0)),
        ],
        out_shape=[
            jax.ShapeDtypeStruct((_B, _L, _D), jnp.float32),
            jax.ShapeDtypeStruct((_B, _L), jnp.int32),
        ],
        scratch_shapes=[
            pltpu.VMEM((2, _LBLK, _D), jnp.float32),
            pltpu.SemaphoreType.DMA((2,)),
            pltpu.SemaphoreType.DMA((2,)),
        ],
    )(tags.astype(jnp.int32), embedding)
    return embed, mask


# speedup vs baseline: 1.1150x; 1.1150x over previous
import jax
import jax.numpy as jnp
from jax.experimental import pallas as pl
from jax.experimental.pallas import tpu as pltpu

_NUM_TAGS = 32
_SHIFT = 128
_L = _NUM_TAGS * _SHIFT
_D = 512
_B = 8
_K = 8
_LBLK = 512
_NBLK = _L // _LBLK
_NS = 4


def _manual_kernel(tags_ref, emb_hbm, out_hbm, mask_ref, buf, in_sem, out_sem):
    tag_ids = jax.lax.broadcasted_iota(jnp.int32, (1, _L), 1) // _SHIFT
    tags = tags_ref[...]
    acc = jnp.zeros((_B, _L), jnp.bool_)
    for k in range(_K):
        acc = acc | (tags[:, k : k + 1] == tag_ids)
    mask_ref[...] = acc.astype(jnp.int32)

    def cp_in(l, s):
        return pltpu.make_async_copy(
            emb_hbm.at[pl.ds(l * _LBLK, _LBLK), :], buf.at[s], in_sem.at[s]
        )

    def cp_out(l, b, s):
        return pltpu.make_async_copy(
            buf.at[s], out_hbm.at[b, pl.ds(l * _LBLK, _LBLK), :], out_sem.at[s]
        )

    cp_in(0, 0).start()
    cp_in(1, 1).start()
    for l in range(_NBLK):
        s = l % _NS
        cp_in(l, s).wait()
        for b in range(_B):
            cp_out(l, b, s).start()
        nl = l + 2
        if nl < _NBLK:
            ol = nl - _NS
            if ol >= 0:
                for b in range(_B):
                    cp_out(ol, b, nl % _NS).wait()
            cp_in(nl, nl % _NS).start()
    for l in range(_NBLK - _NS, _NBLK):
        for b in range(_B):
            cp_out(l, b, l % _NS).wait()


def kernel(tags, embedding):
    embed, mask = pl.pallas_call(
        _manual_kernel,
        in_specs=[
            pl.BlockSpec((_B, _K), lambda: (0, 0)),
            pl.BlockSpec(memory_space=pl.ANY),
        ],
        out_specs=[
            pl.BlockSpec(memory_space=pl.ANY),
            pl.BlockSpec((_B, _L), lambda: (0, 0)),
        ],
        out_shape=[
            jax.ShapeDtypeStruct((_B, _L, _D), jnp.float32),
            jax.ShapeDtypeStruct((_B, _L), jnp.int32),
        ],
        scratch_shapes=[
            pltpu.VMEM((_NS, _LBLK, _D), jnp.float32),
            pltpu.SemaphoreType.DMA((_NS,)),
            pltpu.SemaphoreType.DMA((_NS,)),
        ],
    )(tags.astype(jnp.int32), embedding)
    return embed, mask


# speedup vs baseline: 1.1414x; 1.0237x over previous
import jax
import jax.numpy as jnp
from jax.experimental import pallas as pl

_NUM_TAGS = 32
_SHIFT = 128
_L = _NUM_TAGS * _SHIFT
_D = 512
_B = 8
_K = 8
_LBLK = 512


def _copy_mask_kernel(tags_ref, emb_ref, out_ref, mask_ref):
    l = pl.program_id(0)
    x = emb_ref[...]
    out_ref[...] = jnp.broadcast_to(x[None], (_B, _LBLK, _D))
    base = l * _LBLK
    tag_ids = (base + jax.lax.broadcasted_iota(jnp.int32, (1, _LBLK), 1)) // _SHIFT
    tags = tags_ref[...]
    acc = jnp.zeros((_B, _LBLK), jnp.bool_)
    for k in range(_K):
        acc = acc | (tags[:, k : k + 1] == tag_ids)
    mask_ref[...] = acc.astype(jnp.int32)


def kernel(tags, embedding):
    num_l = _L // _LBLK
    embed, mask = pl.pallas_call(
        _copy_mask_kernel,
        grid=(num_l,),
        in_specs=[
            pl.BlockSpec((_B, _K), lambda l: (0, 0)),
            pl.BlockSpec((_LBLK, _D), lambda l: (l, 0)),
        ],
        out_specs=[
            pl.BlockSpec((_B, _LBLK, _D), lambda l: (0, l, 0)),
            pl.BlockSpec((_B, _LBLK), lambda l: (0, l)),
        ],
        out_shape=[
            jax.ShapeDtypeStruct((_B, _L, _D), jnp.float32),
            jax.ShapeDtypeStruct((_B, _L), jnp.int32),
        ],
    )(tags.astype(jnp.int32), embedding)
    return embed, mask
